# Initial kernel scaffold; baseline (speedup 1.0000x reference)
import jax
import jax.numpy as jnp
from jax.experimental import pallas as pl


def kernel(scores, relevance):
    raise NotImplementedError("write your pallas kernel here")


# speedup vs baseline: 92.0380x; 92.0380x over previous
import numpy as np
import jax
import jax.numpy as jnp
from jax.experimental import pallas as pl
from jax.experimental.pallas import tpu as pltpu

_N = 8192
_B = 128
_K = 64
_Z = 1.0 / (1.0 - float(np.exp(-1.0)))


_R = 8


def _row_kernel(scores_ref, rel_ref, out_ref):
    pid = pl.program_id(0)

    row_i = jax.lax.broadcasted_iota(jnp.int32, (_K, _N), 0)
    col_j = jax.lax.broadcasted_iota(jnp.int32, (_K, _N), 1)
    target_col = row_i + (_N - _K)
    sel = col_j == target_col
    colf = col_j.astype(jnp.float32)
    iota_k = jax.lax.broadcasted_iota(jnp.int32, (_K, 1), 0).astype(jnp.float32)
    w = jnp.exp(iota_k - float(_K - 1))

    acc = jnp.zeros((1, 1), jnp.float32)
    for row in range(_R):
        s = scores_ref[row:row + 1, :]
        r = rel_ref[row:row + 1, :]

        mx = jnp.max(s, axis=1, keepdims=True)
        lse = mx + jnp.log(jnp.sum(jnp.exp(s - mx), axis=1, keepdims=True))

        v = jnp.sum(jnp.where(sel, r, 0.0), axis=1, keepdims=True)

        lt = r < v
        tie = (r == v) & (col_j < target_col)
        cnt = jnp.where(lt, 1.0, 0.0) + jnp.where(tie, 1.0, 0.0)
        ranks = jnp.sum(cnt, axis=1, keepdims=True)

        g = jnp.sum(jnp.where(colf == ranks, s, 0.0), axis=1, keepdims=True)

        dot = jnp.sum(g * w, axis=0, keepdims=True)
        acc = acc + (lse - dot * (1.0 / _Z))

    @pl.when(pid == 0)
    def _():
        out_ref[...] = jnp.zeros_like(out_ref)

    out_ref[...] += acc * (1.0 / _B)


def kernel(scores, relevance):
    out = pl.pallas_call(
        _row_kernel,
        grid=(_B // _R,),
        in_specs=[
            pl.BlockSpec((_R, _N), lambda i: (i, 0)),
            pl.BlockSpec((_R, _N), lambda i: (i, 0)),
        ],
        out_specs=pl.BlockSpec((1, 1), lambda i: (0, 0)),
        out_shape=jax.ShapeDtypeStruct((1, 1), jnp.float32),
    )(scores, relevance)
    return out[0, 0]


# speedup vs baseline: 303.1906x; 3.2942x over previous
import numpy as np
import jax
import jax.numpy as jnp
from jax.experimental import pallas as pl
from jax.experimental.pallas import tpu as pltpu

_N = 8192
_B = 128
_K = 16
_Z = 1.0 / (1.0 - float(np.exp(-1.0)))

_R = 8
_T = 128


def _row_kernel(scores_ref, rel_ref, out_ref):
    pid = pl.program_id(0)

    row_i = jax.lax.broadcasted_iota(jnp.int32, (_K, _T), 0)
    tcol_j = jax.lax.broadcasted_iota(jnp.int32, (_K, _T), 1)
    target_col = row_i + (_T - _K)
    sel = tcol_j == target_col
    colf = jax.lax.broadcasted_iota(jnp.int32, (_K, _N), 1).astype(jnp.float32)
    iota_k = jax.lax.broadcasted_iota(jnp.int32, (_K, 1), 0).astype(jnp.float32)
    w = jnp.exp(iota_k - float(_K - 1))

    acc = jnp.zeros((1, 1), jnp.float32)
    for row in range(_R):
        s = scores_ref[row:row + 1, :]
        r_head = rel_ref[row:row + 1, : _N - _T]
        r_tail = rel_ref[row:row + 1, _N - _T:]

        mx = jnp.max(s, axis=1, keepdims=True)
        lse = mx + jnp.log(jnp.sum(jnp.exp(s - mx), axis=1, keepdims=True))

        v = jnp.sum(jnp.where(sel, r_tail, 0.0), axis=1, keepdims=True)

        head_cnt = jnp.sum(jnp.where(r_head <= v, 1.0, 0.0),
                           axis=1, keepdims=True)
        hit = (r_tail < v) | ((r_tail == v) & (tcol_j < target_col))
        tail_cnt = jnp.sum(jnp.where(hit, 1.0, 0.0), axis=1, keepdims=True)
        ranks = head_cnt + tail_cnt

        g = jnp.sum(jnp.where(colf == ranks, s, 0.0), axis=1, keepdims=True)

        dot = jnp.sum(g * w, axis=0, keepdims=True)
        acc = acc + (lse - dot * (1.0 / _Z))

    @pl.when(pid == 0)
    def _():
        out_ref[...] = jnp.zeros_like(out_ref)

    out_ref[...] += acc * (1.0 / _B)


def kernel(scores, relevance):
    out = pl.pallas_call(
        _row_kernel,
        grid=(_B // _R,),
        in_specs=[
            pl.BlockSpec((_R, _N), lambda i: (i, 0)),
            pl.BlockSpec((_R, _N), lambda i: (i, 0)),
        ],
        out_specs=pl.BlockSpec((1, 1), lambda i: (0, 0)),
        out_shape=jax.ShapeDtypeStruct((1, 1), jnp.float32),
    )(scores, relevance)
    return out[0, 0]
